# Initial kernel scaffold; baseline (speedup 1.0000x reference)
import jax
import jax.numpy as jnp
from jax.experimental import pallas as pl


def kernel(x, edge_index, W1, b1, W2, b2):
    raise NotImplementedError("write your pallas kernel here")


# speedup vs baseline: 8.8694x; 8.8694x over previous
import functools

import jax
import jax.numpy as jnp
from jax import lax
from jax.experimental import pallas as pl
from jax.experimental.pallas import tpu as pltpu
from jax.experimental.pallas import tpu_sc as plsc

N = 10000
D_IN = 128
DH = 16
NCLS = 10
DW = 128

NC = 2
NS = 16
NW = NC * NS
CH = 128
N_PAD = 10112
RPS = N_PAD // NS


@functools.lru_cache(maxsize=None)
def _make_agg(kpw: int):
    @functools.partial(
        pl.kernel,
        out_type=jax.ShapeDtypeStruct((NC, N_PAD, DW), jnp.float32),
        mesh=plsc.VectorSubcoreMesh(core_axis_name="c", subcore_axis_name="s"),
        scratch_types=[
            pltpu.VMEM((kpw, CH), jnp.int32),
            pltpu.VMEM((kpw, CH), jnp.int32),
            pltpu.VMEM((CH, DW), jnp.float32),
            pltpu.VMEM_SHARED((N_PAD, DW), jnp.float32),
        ],
    )
    def agg(table, src3, dst3, out, src_v, dst_v, rows_v, acc):
        c = lax.axis_index("c")
        s = lax.axis_index("s")
        wid = c * NS + s
        r0 = s * RPS
        pltpu.sync_copy(src3.at[wid], src_v)
        pltpu.sync_copy(dst3.at[wid], dst_v)
        pltpu.sync_copy(table.at[pl.ds(r0, RPS)], acc.at[pl.ds(r0, RPS)])
        plsc.subcore_barrier()

        def body(j, carry):
            pltpu.sync_copy(table.at[src_v.at[j]], rows_v)
            pltpu.sync_copy(rows_v, acc.at[dst_v.at[j]], add=True)
            return carry

        lax.fori_loop(0, kpw, body, 0)
        plsc.subcore_barrier()
        pltpu.sync_copy(acc.at[pl.ds(r0, RPS)], out.at[c, pl.ds(r0, RPS)])

    return agg


def _mm_body(x_ref, w_ref, o_ref):
    o_ref[...] = jnp.dot(x_ref[...], w_ref[...],
                         preferred_element_type=jnp.float32)


def _matmul(x, w):
    m, k = x.shape
    n = w.shape[1]
    return pl.pallas_call(
        _mm_body,
        out_shape=jax.ShapeDtypeStruct((m, n), jnp.float32),
    )(x, w)


def _scale_body(d0, d1, g1, dinv_o, h1s_o):
    deg = d0[...] + d1[...] - 1.0
    dinv = lax.rsqrt(jnp.maximum(deg, 1.0))
    dinv_o[...] = dinv
    h1s_o[...] = g1[...] * dinv


def _scale(d0, d1, g1):
    return pl.pallas_call(
        _scale_body,
        out_shape=(jax.ShapeDtypeStruct((N_PAD, DW), jnp.float32),
                   jax.ShapeDtypeStruct((N_PAD, DW), jnp.float32)),
    )(d0, d1, g1)


def _mid_body(q0, q1, h1s, dinv, b1, w2, o):
    a = q0[...] + q1[...] - h1s[...]
    h1 = jnp.maximum(dinv[...] * a + b1[...], 0.0)
    o[...] = jnp.dot(h1, w2[...],
                     preferred_element_type=jnp.float32) * dinv[...]


def _mid(q0, q1, h1s, dinv, b1, w2p):
    return pl.pallas_call(
        _mid_body,
        out_shape=jax.ShapeDtypeStruct((N_PAD, DW), jnp.float32),
    )(q0, q1, h1s, dinv, b1, w2p)


def _out_body(r0, r1, h2s, dinv, b2, o):
    a = r0[...] + r1[...] - h2s[...]
    logits = dinv[...] * a + b2[...]
    col = lax.broadcasted_iota(jnp.int32, logits.shape, 1)
    valid = col < NCLS
    ml = jnp.where(valid, logits, jnp.float32(-1e30))
    m = jnp.max(ml, axis=1, keepdims=True)
    ex = jnp.where(valid, jnp.exp(logits - m), 0.0)
    lse = jnp.log(jnp.sum(ex, axis=1, keepdims=True))
    o[...] = logits - m - lse


def _out(r0, r1, h2s, dinv, b2p):
    return pl.pallas_call(
        _out_body,
        out_shape=jax.ShapeDtypeStruct((N_PAD, DW), jnp.float32),
    )(r0, r1, h2s, dinv, b2p)


def kernel(x, edge_index, W1, b1, W2, b2):
    e = edge_index.shape[1]
    kpw = -(-e // (NW * CH))
    e_pad = NW * CH * kpw

    ei = edge_index.astype(jnp.int32)
    pad = jnp.full((2, e_pad - e), N, dtype=jnp.int32)
    ei = jnp.concatenate([ei, pad], axis=1)
    src3 = ei[0].reshape(NW, kpw, CH)
    dst3 = ei[1].reshape(NW, kpw, CH)

    xp = jnp.zeros((N_PAD, D_IN), jnp.float32).at[:N].set(x)
    ones = jnp.ones((N_PAD, DW), jnp.float32)
    w1p = jnp.zeros((D_IN, DW), jnp.float32).at[:, :DH].set(W1)
    w2p = jnp.zeros((DW, DW), jnp.float32).at[:DH, :NCLS].set(W2)
    b1p = jnp.zeros((1, DW), jnp.float32).at[0, :DH].set(b1)
    b2p = jnp.zeros((1, DW), jnp.float32).at[0, :NCLS].set(b2)

    agg = _make_agg(kpw)

    deg_p = agg(ones, src3, dst3)
    g1 = _matmul(xp, w1p)
    dinv, h1s = _scale(deg_p[0], deg_p[1], g1)
    q = agg(h1s, src3, dst3)
    h2s = _mid(q[0], q[1], h1s, dinv, b1p, w2p)
    r = agg(h2s, src3, dst3)
    out = _out(r[0], r[1], h2s, dinv, b2p)
    return out[:N, :NCLS]
